# baseline (speedup 1.0000x reference)
import functools

import jax
import jax.numpy as jnp
from jax import lax
from jax.experimental import pallas as pl
from jax.experimental.pallas import tpu as pltpu
from jax.experimental.pallas import tpu_sc as plsc

_N_FEATURES = 26
_N_VALUES = 38462
_EMBED = 16
_BATCH = 16384
_N_ROWS = _N_VALUES * _N_FEATURES

_LANES = 16
_G = 128
_NW = 32
_B_PER_W = _BATCH // _NW
_G_PER_F = _B_PER_W // _G

_FULL_BLOCKS = _N_ROWS // 128
_TAIL_ROWS = _N_ROWS - _FULL_BLOCKS * 128
_BLK_PER_W = 245


@functools.cache
def _build_relayout():
    mesh = plsc.VectorSubcoreMesh(core_axis_name="c", subcore_axis_name="s")

    @functools.partial(
        pl.kernel,
        mesh=mesh,
        compiler_params=pltpu.CompilerParams(
            use_tc_tiling_on_sc=True, needs_layout_passes=False
        ),
        out_type=jax.ShapeDtypeStruct((_N_ROWS * _EMBED,), jnp.float32),
        scratch_types=[
            pltpu.VMEM((8, 128), jnp.float32),
            pltpu.VMEM((8, 128), jnp.float32),
            pltpu.VMEM((8, 128), jnp.float32),
            pltpu.VMEM((8, 128), jnp.float32),
            pltpu.VMEM((2048,), jnp.float32),
            pltpu.VMEM((2048,), jnp.float32),
            pltpu.VMEM((_TAIL_ROWS * _EMBED,), jnp.float32),
            pltpu.SemaphoreType.DMA,
            pltpu.SemaphoreType.DMA,
        ],
    )
    def run(tt_hbm, tail_hbm, out_hbm, a0, b0, a1, b1, o0, o1, tbuf, isem, osem):
        wid = lax.axis_index("s") * 2 + lax.axis_index("c")
        gbase = wid * _BLK_PER_W
        lane = lax.iota(jnp.int32, _LANES)
        rowpat = lane * _EMBED

        @pl.when(wid == 0)
        def _tail():
            pltpu.sync_copy(tail_hbm, tbuf)
            pltpu.sync_copy(
                tbuf, out_hbm.at[pl.ds(_FULL_BLOCKS * 2048, _TAIL_ROWS * _EMBED)]
            )

        def blk(j, _):
            g = gbase + j

            @pl.when(g < _FULL_BLOCKS)
            def _do():
                pltpu.sync_copy(tt_hbm.at[pl.ds(0, 8), pl.ds(g * 128, 128)], a0)
                pltpu.sync_copy(tt_hbm.at[pl.ds(8, 8), pl.ds(g * 128, 128)], b0)
                for e in range(8):
                    for c0 in range(0, 128, _LANES):
                        idx = rowpat + (c0 * _EMBED + e)
                        plsc.store_scatter(o0, [idx], a0[e, pl.ds(c0, _LANES)])
                        plsc.store_scatter(o0, [idx + 8], b0[e, pl.ds(c0, _LANES)])
                pltpu.sync_copy(o0, out_hbm.at[pl.ds(g * 2048, 2048)])

            return _

        lax.fori_loop(0, _BLK_PER_W, blk, None)

    return run


@functools.cache
def _build_gather():
    mesh = plsc.VectorSubcoreMesh(core_axis_name="c", subcore_axis_name="s")

    @functools.partial(
        pl.kernel,
        mesh=mesh,
        compiler_params=pltpu.CompilerParams(
            use_tc_tiling_on_sc=False, needs_layout_passes=False
        ),
        out_type=jax.ShapeDtypeStruct((_N_FEATURES, _EMBED * _BATCH), jnp.float32),
        scratch_types=[
            pltpu.VMEM((_N_FEATURES, _B_PER_W), jnp.int32),
            pltpu.VMEM((_B_PER_W, _EMBED), jnp.float32),
            pltpu.VMEM((_B_PER_W, _EMBED), jnp.float32),
            pltpu.VMEM((_EMBED * _B_PER_W,), jnp.float32),
            pltpu.SemaphoreType.DMA,
        ],
    )
    def run(xt_hbm, table_hbm, out_hbm, idx_all, rows_a, rows_b, rows_t, sem):
        wid = lax.axis_index("s") * 2 + lax.axis_index("c")
        b0 = wid * _B_PER_W

        pltpu.sync_copy(xt_hbm.at[:, pl.ds(b0, _B_PER_W)], idx_all)

        lane = lax.iota(jnp.int32, _LANES)
        lanepat = (
            lax.shift_right_logical(lane, 3) * (_B_PER_W * 8)
            + (lane & 7) * 128
        )

        def fire(f, dst):
            seg = table_hbm.at[pl.ds(f * _N_VALUES, _N_VALUES)]
            idx_row = idx_all.at[f]
            for g in range(_G_PER_F):
                pltpu.async_copy(
                    seg.at[idx_row.at[pl.ds(g * _G, _G)]],
                    dst.at[pl.ds(g * _G, _G)],
                    sem,
                )

        def drain(dst):
            for g in range(_G_PER_F):
                pltpu.make_async_copy(
                    table_hbm.at[pl.ds(0, _G)], dst.at[pl.ds(g * _G, _G)], sem
                ).wait()

        def transpose_store(f, src):
            def tr16(k, _):
                rbase = k * _LANES
                cb = (rbase // 128) * 1024 + rbase % 128
                for i in range(_LANES):
                    v = src[rbase + i, :]
                    plsc.store_scatter(rows_t, [lanepat + (cb + i)], v)
                return _

            lax.fori_loop(0, _B_PER_W // _LANES, tr16, None)
            for et in range(_EMBED // 8):
                pltpu.sync_copy(
                    rows_t.at[pl.ds(et * (_B_PER_W * 8), _B_PER_W * 8)],
                    out_hbm.at[
                        f,
                        pl.ds(et * (_BATCH * 8) + wid * (_B_PER_W * 8), _B_PER_W * 8),
                    ],
                )

        fire(0, rows_a)

        def pair(p, _):
            fa = 2 * p
            drain(rows_a)
            fire(fa + 1, rows_b)
            transpose_store(fa, rows_a)
            drain(rows_b)

            @pl.when(p < _N_FEATURES // 2 - 1)
            def _fire_next():
                fire(fa + 2, rows_a)

            transpose_store(fa + 1, rows_b)
            return _

        lax.fori_loop(0, _N_FEATURES // 2, pair, None)

    return run


def kernel(x, table):
    tt = jnp.transpose(table)
    tail = table[_FULL_BLOCKS * 128 :, :].reshape(-1)
    tbl_lin = _build_relayout()(tt, tail).reshape(_N_ROWS, _EMBED)

    xt = jnp.transpose(x)
    out_k = _build_gather()(xt, tbl_lin)
    k5 = out_k.reshape(_N_FEATURES, 2, _BATCH // 128, 8, 128)
    t = jnp.transpose(k5, (2, 4, 0, 1, 3))
    return t.reshape(_BATCH, _N_FEATURES, _EMBED)


# speedup vs baseline: 1.7694x; 1.7694x over previous
import functools

import jax
import jax.numpy as jnp
from jax import lax
from jax.experimental import pallas as pl
from jax.experimental.pallas import tpu as pltpu
from jax.experimental.pallas import tpu_sc as plsc

_N_FEATURES = 26
_N_VALUES = 38462
_EMBED = 16
_BATCH = 16384
_N_ROWS = _N_VALUES * _N_FEATURES

_LANES = 16
_G = 128
_NW = 32
_B_PER_W = _BATCH // _NW
_G_PER_F = _B_PER_W // _G

_FULL_BLOCKS = _N_ROWS // 128
_TAIL_ROWS = _N_ROWS - _FULL_BLOCKS * 128
_BLK_PER_W = 244
_EXTRA_BLOCKS = _FULL_BLOCKS - _BLK_PER_W * _NW


@functools.cache
def _build_relayout():
    mesh = plsc.VectorSubcoreMesh(core_axis_name="c", subcore_axis_name="s")

    @functools.partial(
        pl.kernel,
        mesh=mesh,
        compiler_params=pltpu.CompilerParams(
            use_tc_tiling_on_sc=True, needs_layout_passes=False
        ),
        out_type=jax.ShapeDtypeStruct((_N_ROWS * _EMBED,), jnp.float32),
        scratch_types=[
            pltpu.VMEM((8, 128), jnp.float32),
            pltpu.VMEM((8, 128), jnp.float32),
            pltpu.VMEM((8, 128), jnp.float32),
            pltpu.VMEM((8, 128), jnp.float32),
            pltpu.VMEM((2048,), jnp.float32),
            pltpu.VMEM((2048,), jnp.float32),
            pltpu.VMEM((_TAIL_ROWS * _EMBED,), jnp.float32),
            pltpu.SemaphoreType.DMA,
            pltpu.SemaphoreType.DMA,
        ],
    )
    def run(tt_hbm, tail_hbm, out_hbm, a0, b0, a1, b1, o0, o1, tbuf, isem, osem):
        wid = lax.axis_index("s") * 2 + lax.axis_index("c")
        gbase = wid * _BLK_PER_W
        lane = lax.iota(jnp.int32, _LANES)
        rowpat = lane * _EMBED

        @pl.when(wid == 0)
        def _tail():
            pltpu.sync_copy(tail_hbm, tbuf)
            pltpu.sync_copy(
                tbuf, out_hbm.at[pl.ds(_FULL_BLOCKS * 2048, _TAIL_ROWS * _EMBED)]
            )

        def fire_in(j, bufa, bufb, sem):
            g = gbase + j
            pltpu.async_copy(tt_hbm.at[pl.ds(0, 8), pl.ds(g * 128, 128)], bufa, sem)
            pltpu.async_copy(tt_hbm.at[pl.ds(8, 8), pl.ds(g * 128, 128)], bufb, sem)

        def drain_in(bufa, bufb, sem):
            pltpu.make_async_copy(tt_hbm.at[pl.ds(0, 8), pl.ds(0, 128)], bufa, sem).wait()
            pltpu.make_async_copy(tt_hbm.at[pl.ds(0, 8), pl.ds(0, 128)], bufb, sem).wait()

        def compute_out(g, bufa, bufb, obuf):
            for e in range(8):
                for c0 in range(0, 128, _LANES):
                    idx = rowpat + (c0 * _EMBED + e)
                    plsc.store_scatter(obuf, [idx], bufa[e, pl.ds(c0, _LANES)])
                    plsc.store_scatter(obuf, [idx + 8], bufb[e, pl.ds(c0, _LANES)])
            pltpu.sync_copy(obuf, out_hbm.at[pl.ds(g * 2048, 2048)])

        fire_in(0, a0, b0, isem)
        fire_in(1, a1, b1, osem)

        def pair(p, _):
            ja = 2 * p
            drain_in(a0, b0, isem)
            compute_out(gbase + ja, a0, b0, o0)
            fire_in(ja + 2, a0, b0, isem)
            drain_in(a1, b1, osem)
            compute_out(gbase + ja + 1, a1, b1, o1)
            fire_in(ja + 3, a1, b1, osem)
            return _

        lax.fori_loop(0, _BLK_PER_W // 2, pair, None)

        drain_in(a0, b0, isem)
        drain_in(a1, b1, osem)

        @pl.when(wid < _EXTRA_BLOCKS)
        def _extra():
            g = _BLK_PER_W * _NW + wid
            pltpu.sync_copy(tt_hbm.at[pl.ds(0, 8), pl.ds(g * 128, 128)], a0)
            pltpu.sync_copy(tt_hbm.at[pl.ds(8, 8), pl.ds(g * 128, 128)], b0)
            compute_out(g, a0, b0, o0)

    return run


@functools.cache
def _build_gather():
    mesh = plsc.VectorSubcoreMesh(core_axis_name="c", subcore_axis_name="s")

    @functools.partial(
        pl.kernel,
        mesh=mesh,
        compiler_params=pltpu.CompilerParams(
            use_tc_tiling_on_sc=False, needs_layout_passes=False
        ),
        out_type=jax.ShapeDtypeStruct((_N_FEATURES, _EMBED * _BATCH), jnp.float32),
        scratch_types=[
            pltpu.VMEM((_N_FEATURES, _B_PER_W), jnp.int32),
            pltpu.VMEM((_B_PER_W, _EMBED), jnp.float32),
            pltpu.VMEM((_B_PER_W, _EMBED), jnp.float32),
            pltpu.VMEM((_EMBED * _B_PER_W,), jnp.float32),
            pltpu.SemaphoreType.DMA,
        ],
    )
    def run(xt_hbm, table_hbm, out_hbm, idx_all, rows_a, rows_b, rows_t, sem):
        wid = lax.axis_index("s") * 2 + lax.axis_index("c")
        b0 = wid * _B_PER_W

        pltpu.sync_copy(xt_hbm.at[:, pl.ds(b0, _B_PER_W)], idx_all)

        lane = lax.iota(jnp.int32, _LANES)
        lanepat = (
            lax.shift_right_logical(lane, 3) * (_B_PER_W * 8)
            + (lane & 7) * 128
        )

        def fire(f, dst):
            seg = table_hbm.at[pl.ds(f * _N_VALUES, _N_VALUES)]
            idx_row = idx_all.at[f]
            for g in range(_G_PER_F):
                pltpu.async_copy(
                    seg.at[idx_row.at[pl.ds(g * _G, _G)]],
                    dst.at[pl.ds(g * _G, _G)],
                    sem,
                )

        def drain(dst):
            for g in range(_G_PER_F):
                pltpu.make_async_copy(
                    table_hbm.at[pl.ds(0, _G)], dst.at[pl.ds(g * _G, _G)], sem
                ).wait()

        def transpose_store(f, src):
            def tr16(k, _):
                rbase = k * _LANES
                cb = (rbase // 128) * 1024 + rbase % 128
                for i in range(_LANES):
                    v = src[rbase + i, :]
                    plsc.store_scatter(rows_t, [lanepat + (cb + i)], v)
                return _

            lax.fori_loop(0, _B_PER_W // _LANES, tr16, None)
            for et in range(_EMBED // 8):
                pltpu.sync_copy(
                    rows_t.at[pl.ds(et * (_B_PER_W * 8), _B_PER_W * 8)],
                    out_hbm.at[
                        f,
                        pl.ds(et * (_BATCH * 8) + wid * (_B_PER_W * 8), _B_PER_W * 8),
                    ],
                )

        fire(0, rows_a)

        def pair(p, _):
            fa = 2 * p
            drain(rows_a)
            fire(fa + 1, rows_b)
            transpose_store(fa, rows_a)
            drain(rows_b)

            @pl.when(p < _N_FEATURES // 2 - 1)
            def _fire_next():
                fire(fa + 2, rows_a)

            transpose_store(fa + 1, rows_b)
            return _

        lax.fori_loop(0, _N_FEATURES // 2, pair, None)

    return run


def kernel(x, table):
    tt = jnp.transpose(table)
    tail = table[_FULL_BLOCKS * 128 :, :].reshape(-1)
    tbl_lin = _build_relayout()(tt, tail).reshape(_N_ROWS, _EMBED)

    xt = jnp.transpose(x)
    out_k = _build_gather()(xt, tbl_lin)
    k5 = out_k.reshape(_N_FEATURES, 2, _BATCH // 128, 8, 128)
    t = jnp.transpose(k5, (2, 4, 0, 1, 3))
    return t.reshape(_BATCH, _N_FEATURES, _EMBED)


# speedup vs baseline: 1.8521x; 1.0468x over previous
import functools

import jax
import jax.numpy as jnp
from jax import lax
from jax.experimental import pallas as pl
from jax.experimental.pallas import tpu as pltpu
from jax.experimental.pallas import tpu_sc as plsc

_N_FEATURES = 26
_N_VALUES = 38462
_EMBED = 16
_BATCH = 16384
_N_ROWS = _N_VALUES * _N_FEATURES

_LANES = 16
_G = 128
_NW = 32
_B_PER_W = _BATCH // _NW
_G_PER_F = _B_PER_W // _G

_FULL_BLOCKS = _N_ROWS // 128
_TAIL_ROWS = _N_ROWS - _FULL_BLOCKS * 128
_BLK_PER_W = 244
_EXTRA_BLOCKS = _FULL_BLOCKS - _BLK_PER_W * _NW


@functools.cache
def _build_relayout():
    mesh = plsc.VectorSubcoreMesh(core_axis_name="c", subcore_axis_name="s")

    @functools.partial(
        pl.kernel,
        mesh=mesh,
        compiler_params=pltpu.CompilerParams(
            use_tc_tiling_on_sc=True, needs_layout_passes=False
        ),
        out_type=jax.ShapeDtypeStruct((_N_ROWS * _EMBED,), jnp.float32),
        scratch_types=[
            pltpu.VMEM((8, 128), jnp.float32),
            pltpu.VMEM((8, 128), jnp.float32),
            pltpu.VMEM((8, 128), jnp.float32),
            pltpu.VMEM((8, 128), jnp.float32),
            pltpu.VMEM((2048,), jnp.float32),
            pltpu.VMEM((2048,), jnp.float32),
            pltpu.VMEM((_TAIL_ROWS * _EMBED,), jnp.float32),
            pltpu.SemaphoreType.DMA,
            pltpu.SemaphoreType.DMA,
            pltpu.SemaphoreType.DMA,
            pltpu.SemaphoreType.DMA,
        ],
    )
    def run(
        tt_hbm, tail_hbm, out_hbm, a0, b0, a1, b1, o0, o1, tbuf, isem, osem, ws0, ws1
    ):
        wid = lax.axis_index("s") * 2 + lax.axis_index("c")
        gbase = wid * _BLK_PER_W
        lane = lax.iota(jnp.int32, _LANES)
        rowpat = lane * _EMBED

        @pl.when(wid == 0)
        def _tail():
            pltpu.sync_copy(tail_hbm, tbuf)
            pltpu.sync_copy(
                tbuf, out_hbm.at[pl.ds(_FULL_BLOCKS * 2048, _TAIL_ROWS * _EMBED)]
            )

        def fire_in(j, bufa, bufb, sem):
            g = gbase + j
            pltpu.async_copy(tt_hbm.at[pl.ds(0, 8), pl.ds(g * 128, 128)], bufa, sem)
            pltpu.async_copy(tt_hbm.at[pl.ds(8, 8), pl.ds(g * 128, 128)], bufb, sem)

        def drain_in(bufa, bufb, sem):
            pltpu.make_async_copy(tt_hbm.at[pl.ds(0, 8), pl.ds(0, 128)], bufa, sem).wait()
            pltpu.make_async_copy(tt_hbm.at[pl.ds(0, 8), pl.ds(0, 128)], bufb, sem).wait()

        def compute(bufa, bufb, obuf):
            for e in range(8):
                for c0 in range(0, 128, _LANES):
                    idx = rowpat + (c0 * _EMBED + e)
                    plsc.store_scatter(obuf, [idx], bufa[e, pl.ds(c0, _LANES)])
                    plsc.store_scatter(obuf, [idx + 8], bufb[e, pl.ds(c0, _LANES)])

        def fire_out(g, obuf, sem):
            pltpu.async_copy(obuf, out_hbm.at[pl.ds(g * 2048, 2048)], sem)

        def drain_out(sem):
            pltpu.make_async_copy(o0, out_hbm.at[pl.ds(0, 2048)], sem).wait()

        def compute_out(g, bufa, bufb, obuf):
            compute(bufa, bufb, obuf)
            pltpu.sync_copy(obuf, out_hbm.at[pl.ds(g * 2048, 2048)])

        fire_in(0, a0, b0, isem)
        fire_in(1, a1, b1, osem)

        drain_in(a0, b0, isem)
        compute(a0, b0, o0)
        fire_out(gbase, o0, ws0)
        fire_in(2, a0, b0, isem)
        drain_in(a1, b1, osem)
        compute(a1, b1, o1)
        fire_out(gbase + 1, o1, ws1)
        fire_in(3, a1, b1, osem)

        def pair(p, _):
            ja = 2 * p
            drain_in(a0, b0, isem)
            drain_out(ws0)
            compute(a0, b0, o0)
            fire_out(gbase + ja, o0, ws0)
            fire_in(ja + 2, a0, b0, isem)
            drain_in(a1, b1, osem)
            drain_out(ws1)
            compute(a1, b1, o1)
            fire_out(gbase + ja + 1, o1, ws1)
            fire_in(ja + 3, a1, b1, osem)
            return _

        lax.fori_loop(1, _BLK_PER_W // 2, pair, None)

        drain_in(a0, b0, isem)
        drain_in(a1, b1, osem)
        drain_out(ws0)
        drain_out(ws1)

        @pl.when(wid < _EXTRA_BLOCKS)
        def _extra():
            g = _BLK_PER_W * _NW + wid
            pltpu.sync_copy(tt_hbm.at[pl.ds(0, 8), pl.ds(g * 128, 128)], a0)
            pltpu.sync_copy(tt_hbm.at[pl.ds(8, 8), pl.ds(g * 128, 128)], b0)
            compute_out(g, a0, b0, o0)

    return run


@functools.cache
def _build_gather():
    mesh = plsc.VectorSubcoreMesh(core_axis_name="c", subcore_axis_name="s")

    @functools.partial(
        pl.kernel,
        mesh=mesh,
        compiler_params=pltpu.CompilerParams(
            use_tc_tiling_on_sc=False, needs_layout_passes=False
        ),
        out_type=jax.ShapeDtypeStruct((_N_FEATURES, _EMBED * _BATCH), jnp.float32),
        scratch_types=[
            pltpu.VMEM((_N_FEATURES, _B_PER_W), jnp.int32),
            pltpu.VMEM((_B_PER_W, _EMBED), jnp.float32),
            pltpu.VMEM((_B_PER_W, _EMBED), jnp.float32),
            pltpu.VMEM((_EMBED * _B_PER_W,), jnp.float32),
            pltpu.SemaphoreType.DMA,
        ],
    )
    def run(xt_hbm, table_hbm, out_hbm, idx_all, rows_a, rows_b, rows_t, sem):
        wid = lax.axis_index("s") * 2 + lax.axis_index("c")
        b0 = wid * _B_PER_W

        pltpu.sync_copy(xt_hbm.at[:, pl.ds(b0, _B_PER_W)], idx_all)

        lane = lax.iota(jnp.int32, _LANES)
        lanepat = (
            lax.shift_right_logical(lane, 3) * (_B_PER_W * 8)
            + (lane & 7) * 128
        )

        def fire(f, dst):
            seg = table_hbm.at[pl.ds(f * _N_VALUES, _N_VALUES)]
            idx_row = idx_all.at[f]
            for g in range(_G_PER_F):
                pltpu.async_copy(
                    seg.at[idx_row.at[pl.ds(g * _G, _G)]],
                    dst.at[pl.ds(g * _G, _G)],
                    sem,
                )

        def drain(dst):
            for g in range(_G_PER_F):
                pltpu.make_async_copy(
                    table_hbm.at[pl.ds(0, _G)], dst.at[pl.ds(g * _G, _G)], sem
                ).wait()

        def transpose_store(f, src):
            def tr16(k, _):
                rbase = k * _LANES
                cb = (rbase // 128) * 1024 + rbase % 128
                for i in range(_LANES):
                    v = src[rbase + i, :]
                    plsc.store_scatter(rows_t, [lanepat + (cb + i)], v)
                return _

            lax.fori_loop(0, _B_PER_W // _LANES, tr16, None)
            for et in range(_EMBED // 8):
                pltpu.sync_copy(
                    rows_t.at[pl.ds(et * (_B_PER_W * 8), _B_PER_W * 8)],
                    out_hbm.at[
                        f,
                        pl.ds(et * (_BATCH * 8) + wid * (_B_PER_W * 8), _B_PER_W * 8),
                    ],
                )

        fire(0, rows_a)

        def pair(p, _):
            fa = 2 * p
            drain(rows_a)
            fire(fa + 1, rows_b)
            transpose_store(fa, rows_a)
            drain(rows_b)

            @pl.when(p < _N_FEATURES // 2 - 1)
            def _fire_next():
                fire(fa + 2, rows_a)

            transpose_store(fa + 1, rows_b)
            return _

        lax.fori_loop(0, _N_FEATURES // 2, pair, None)

    return run


def kernel(x, table):
    tt = jnp.transpose(table)
    tail = table[_FULL_BLOCKS * 128 :, :].reshape(-1)
    tbl_lin = _build_relayout()(tt, tail).reshape(_N_ROWS, _EMBED)

    xt = jnp.transpose(x)
    out_k = _build_gather()(xt, tbl_lin)
    k5 = out_k.reshape(_N_FEATURES, 2, _BATCH // 128, 8, 128)
    t = jnp.transpose(k5, (2, 4, 0, 1, 3))
    return t.reshape(_BATCH, _N_FEATURES, _EMBED)


# speedup vs baseline: 1.9115x; 1.0321x over previous
import functools

import jax
import jax.numpy as jnp
from jax import lax
from jax.experimental import pallas as pl
from jax.experimental.pallas import tpu as pltpu
from jax.experimental.pallas import tpu_sc as plsc

_N_FEATURES = 26
_N_VALUES = 38462
_EMBED = 16
_BATCH = 16384
_N_ROWS = _N_VALUES * _N_FEATURES

_LANES = 16
_G = 128
_NW = 32
_B_PER_W = _BATCH // _NW
_G_PER_F = _B_PER_W // _G

_FULL_BLOCKS = _N_ROWS // 128
_TAIL_ROWS = _N_ROWS - _FULL_BLOCKS * 128
_BLK_PER_W = 244
_EXTRA_BLOCKS = _FULL_BLOCKS - _BLK_PER_W * _NW


@functools.cache
def _build_relayout():
    mesh = plsc.VectorSubcoreMesh(core_axis_name="c", subcore_axis_name="s")

    @functools.partial(
        pl.kernel,
        mesh=mesh,
        compiler_params=pltpu.CompilerParams(
            use_tc_tiling_on_sc=True, needs_layout_passes=False
        ),
        out_type=jax.ShapeDtypeStruct((_N_ROWS * _EMBED,), jnp.float32),
        scratch_types=[
            pltpu.VMEM((8, 128), jnp.float32),
            pltpu.VMEM((8, 128), jnp.float32),
            pltpu.VMEM((8, 128), jnp.float32),
            pltpu.VMEM((8, 128), jnp.float32),
            pltpu.VMEM((2048,), jnp.float32),
            pltpu.VMEM((2048,), jnp.float32),
            pltpu.VMEM((_TAIL_ROWS * _EMBED,), jnp.float32),
            pltpu.SemaphoreType.DMA,
            pltpu.SemaphoreType.DMA,
            pltpu.SemaphoreType.DMA,
            pltpu.SemaphoreType.DMA,
        ],
    )
    def run(
        tt_hbm, tail_hbm, out_hbm, a0, b0, a1, b1, o0, o1, tbuf, isem, osem, ws0, ws1
    ):
        wid = lax.axis_index("s") * 2 + lax.axis_index("c")
        gbase = wid * _BLK_PER_W
        lane = lax.iota(jnp.int32, _LANES)
        rowpat = lane * _EMBED

        @pl.when(wid == 0)
        def _tail():
            pltpu.sync_copy(tail_hbm, tbuf)
            pltpu.sync_copy(
                tbuf, out_hbm.at[pl.ds(_FULL_BLOCKS * 2048, _TAIL_ROWS * _EMBED)]
            )

        def fire_in(j, bufa, bufb, sem):
            g = gbase + j
            pltpu.async_copy(tt_hbm.at[pl.ds(0, 8), pl.ds(g * 128, 128)], bufa, sem)
            pltpu.async_copy(tt_hbm.at[pl.ds(8, 8), pl.ds(g * 128, 128)], bufb, sem)

        def drain_in(bufa, bufb, sem):
            pltpu.make_async_copy(tt_hbm.at[pl.ds(0, 8), pl.ds(0, 128)], bufa, sem).wait()
            pltpu.make_async_copy(tt_hbm.at[pl.ds(0, 8), pl.ds(0, 128)], bufb, sem).wait()

        def compute(bufa, bufb, obuf):
            for e in range(8):
                for c0 in range(0, 128, _LANES):
                    idx = rowpat + (c0 * _EMBED + e)
                    plsc.store_scatter(obuf, [idx], bufa[e, pl.ds(c0, _LANES)])
                    plsc.store_scatter(obuf, [idx + 8], bufb[e, pl.ds(c0, _LANES)])

        def fire_out(g, obuf, sem):
            pltpu.async_copy(obuf, out_hbm.at[pl.ds(g * 2048, 2048)], sem)

        def drain_out(sem):
            pltpu.make_async_copy(o0, out_hbm.at[pl.ds(0, 2048)], sem).wait()

        def compute_out(g, bufa, bufb, obuf):
            compute(bufa, bufb, obuf)
            pltpu.sync_copy(obuf, out_hbm.at[pl.ds(g * 2048, 2048)])

        fire_in(0, a0, b0, isem)
        fire_in(1, a1, b1, osem)

        drain_in(a0, b0, isem)
        compute(a0, b0, o0)
        fire_out(gbase, o0, ws0)
        fire_in(2, a0, b0, isem)
        drain_in(a1, b1, osem)
        compute(a1, b1, o1)
        fire_out(gbase + 1, o1, ws1)
        fire_in(3, a1, b1, osem)

        def pair(p, _):
            ja = 2 * p
            drain_in(a0, b0, isem)
            drain_out(ws0)
            compute(a0, b0, o0)
            fire_out(gbase + ja, o0, ws0)
            fire_in(ja + 2, a0, b0, isem)
            drain_in(a1, b1, osem)
            drain_out(ws1)
            compute(a1, b1, o1)
            fire_out(gbase + ja + 1, o1, ws1)
            fire_in(ja + 3, a1, b1, osem)
            return _

        lax.fori_loop(1, _BLK_PER_W // 2, pair, None)

        drain_in(a0, b0, isem)
        drain_in(a1, b1, osem)
        drain_out(ws0)
        drain_out(ws1)

        @pl.when(wid < _EXTRA_BLOCKS)
        def _extra():
            g = _BLK_PER_W * _NW + wid
            pltpu.sync_copy(tt_hbm.at[pl.ds(0, 8), pl.ds(g * 128, 128)], a0)
            pltpu.sync_copy(tt_hbm.at[pl.ds(8, 8), pl.ds(g * 128, 128)], b0)
            compute_out(g, a0, b0, o0)

    return run


@functools.cache
def _build_gather():
    mesh = plsc.VectorSubcoreMesh(core_axis_name="c", subcore_axis_name="s")

    @functools.partial(
        pl.kernel,
        mesh=mesh,
        compiler_params=pltpu.CompilerParams(
            use_tc_tiling_on_sc=False, needs_layout_passes=False
        ),
        out_type=jax.ShapeDtypeStruct((_N_FEATURES, _EMBED * _BATCH), jnp.float32),
        scratch_types=[
            pltpu.VMEM((_N_FEATURES, _B_PER_W), jnp.int32),
            pltpu.VMEM((_B_PER_W, _EMBED), jnp.float32),
            pltpu.VMEM((_B_PER_W, _EMBED), jnp.float32),
            pltpu.VMEM((_EMBED * _B_PER_W,), jnp.float32),
            pltpu.VMEM((_EMBED * _B_PER_W,), jnp.float32),
            pltpu.SemaphoreType.DMA,
            pltpu.SemaphoreType.DMA,
            pltpu.SemaphoreType.DMA,
        ],
    )
    def run(
        xt_hbm, table_hbm, out_hbm, idx_all, rows_a, rows_b, t0, t1, sem, os0, os1
    ):
        wid = lax.axis_index("s") * 2 + lax.axis_index("c")
        b0 = wid * _B_PER_W

        pltpu.sync_copy(xt_hbm.at[:, pl.ds(b0, _B_PER_W)], idx_all)

        lane = lax.iota(jnp.int32, _LANES)
        lanepat = (
            lax.shift_right_logical(lane, 3) * (_B_PER_W * 8)
            + (lane & 7) * 128
        )

        def fire(f, dst):
            seg = table_hbm.at[pl.ds(f * _N_VALUES, _N_VALUES)]
            idx_row = idx_all.at[f]
            for g in range(_G_PER_F):
                pltpu.async_copy(
                    seg.at[idx_row.at[pl.ds(g * _G, _G)]],
                    dst.at[pl.ds(g * _G, _G)],
                    sem,
                )

        def drain(dst):
            for g in range(_G_PER_F):
                pltpu.make_async_copy(
                    table_hbm.at[pl.ds(0, _G)], dst.at[pl.ds(g * _G, _G)], sem
                ).wait()

        def transpose(src, dst):
            def tr16(k, _):
                rbase = k * _LANES
                cb = (rbase // 128) * 1024 + rbase % 128
                for i in range(_LANES):
                    v = src[rbase + i, :]
                    plsc.store_scatter(dst, [lanepat + (cb + i)], v)
                return _

            lax.fori_loop(0, _B_PER_W // _LANES, tr16, None)

        def fire_out(f, src, osem):
            for et in range(_EMBED // 8):
                pltpu.async_copy(
                    src.at[pl.ds(et * (_B_PER_W * 8), _B_PER_W * 8)],
                    out_hbm.at[
                        f,
                        pl.ds(et * (_BATCH * 8) + wid * (_B_PER_W * 8), _B_PER_W * 8),
                    ],
                    osem,
                )

        def drain_out(osem):
            for et in range(_EMBED // 8):
                pltpu.make_async_copy(
                    t0.at[pl.ds(0, _B_PER_W * 8)],
                    out_hbm.at[0, pl.ds(0, _B_PER_W * 8)],
                    osem,
                ).wait()

        fire(0, rows_a)

        drain(rows_a)
        fire(1, rows_b)
        transpose(rows_a, t0)
        fire_out(0, t0, os0)
        drain(rows_b)
        fire(2, rows_a)
        transpose(rows_b, t1)
        fire_out(1, t1, os1)

        def pair(p, _):
            fa = 2 * p
            drain(rows_a)
            fire(fa + 1, rows_b)
            drain_out(os0)
            transpose(rows_a, t0)
            fire_out(fa, t0, os0)
            drain(rows_b)

            @pl.when(p < _N_FEATURES // 2 - 1)
            def _fire_next():
                fire(fa + 2, rows_a)

            drain_out(os1)
            transpose(rows_b, t1)
            fire_out(fa + 1, t1, os1)
            return _

        lax.fori_loop(1, _N_FEATURES // 2, pair, None)
        drain_out(os0)
        drain_out(os1)

    return run


def kernel(x, table):
    tt = jnp.transpose(table)
    tail = table[_FULL_BLOCKS * 128 :, :].reshape(-1)
    tbl_lin = _build_relayout()(tt, tail).reshape(_N_ROWS, _EMBED)

    xt = jnp.transpose(x)
    out_k = _build_gather()(xt, tbl_lin)
    k5 = out_k.reshape(_N_FEATURES, 2, _BATCH // 128, 8, 128)
    t = jnp.transpose(k5, (2, 4, 0, 1, 3))
    return t.reshape(_BATCH, _N_FEATURES, _EMBED)


# speedup vs baseline: 2.1071x; 1.1023x over previous
import functools

import jax
import jax.numpy as jnp
from jax import lax
from jax.experimental import pallas as pl
from jax.experimental.pallas import tpu as pltpu
from jax.experimental.pallas import tpu_sc as plsc

_N_FEATURES = 26
_N_VALUES = 38462
_EMBED = 16
_BATCH = 16384
_N_ROWS = _N_VALUES * _N_FEATURES

_LANES = 16
_G = 128
_NW = 32
_B_PER_W = _BATCH // _NW
_G_PER_F = _B_PER_W // _G

_FULL_BLOCKS = _N_ROWS // 128
_TAIL_ROWS = _N_ROWS - _FULL_BLOCKS * 128
_BLK_PER_W = 244
_EXTRA_BLOCKS = _FULL_BLOCKS - _BLK_PER_W * _NW


@functools.cache
def _build_relayout():
    mesh = plsc.VectorSubcoreMesh(core_axis_name="c", subcore_axis_name="s")

    @functools.partial(
        pl.kernel,
        mesh=mesh,
        compiler_params=pltpu.CompilerParams(
            use_tc_tiling_on_sc=True, needs_layout_passes=False
        ),
        out_type=jax.ShapeDtypeStruct((_N_ROWS * _EMBED,), jnp.float32),
        scratch_types=[
            pltpu.VMEM((8, 128), jnp.float32),
            pltpu.VMEM((8, 128), jnp.float32),
            pltpu.VMEM((8, 128), jnp.float32),
            pltpu.VMEM((8, 128), jnp.float32),
            pltpu.VMEM((2048,), jnp.float32),
            pltpu.VMEM((2048,), jnp.float32),
            pltpu.VMEM((_TAIL_ROWS * _EMBED,), jnp.float32),
            pltpu.SemaphoreType.DMA,
            pltpu.SemaphoreType.DMA,
            pltpu.SemaphoreType.DMA,
            pltpu.SemaphoreType.DMA,
        ],
    )
    def run(
        tt_hbm, tail_hbm, out_hbm, a0, b0, a1, b1, o0, o1, tbuf, isem, osem, ws0, ws1
    ):
        wid = lax.axis_index("s") * 2 + lax.axis_index("c")
        gbase = wid * _BLK_PER_W
        lane = lax.iota(jnp.int32, _LANES)
        rowpat = lane * _EMBED

        @pl.when(wid == 0)
        def _tail():
            pltpu.sync_copy(tail_hbm, tbuf)
            pltpu.sync_copy(
                tbuf, out_hbm.at[pl.ds(_FULL_BLOCKS * 2048, _TAIL_ROWS * _EMBED)]
            )

        def fire_in(j, bufa, bufb, sem):
            g = gbase + j
            pltpu.async_copy(tt_hbm.at[pl.ds(0, 8), pl.ds(g * 128, 128)], bufa, sem)
            pltpu.async_copy(tt_hbm.at[pl.ds(8, 8), pl.ds(g * 128, 128)], bufb, sem)

        def drain_in(bufa, bufb, sem):
            pltpu.make_async_copy(tt_hbm.at[pl.ds(0, 8), pl.ds(0, 128)], bufa, sem).wait()
            pltpu.make_async_copy(tt_hbm.at[pl.ds(0, 8), pl.ds(0, 128)], bufb, sem).wait()

        def compute(bufa, bufb, obuf):
            @plsc.parallel_loop(0, 64, unroll=8)
            def _iter(i):
                e = i // 8
                c0 = (i % 8) * _LANES
                idx = rowpat + (c0 * _EMBED + e)
                plsc.store_scatter(obuf, [idx], bufa[e, pl.ds(c0, _LANES)])
                plsc.store_scatter(obuf, [idx + 8], bufb[e, pl.ds(c0, _LANES)])

        def fire_out(g, obuf, sem):
            pltpu.async_copy(obuf, out_hbm.at[pl.ds(g * 2048, 2048)], sem)

        def drain_out(sem):
            pltpu.make_async_copy(o0, out_hbm.at[pl.ds(0, 2048)], sem).wait()

        def compute_out(g, bufa, bufb, obuf):
            compute(bufa, bufb, obuf)
            pltpu.sync_copy(obuf, out_hbm.at[pl.ds(g * 2048, 2048)])

        fire_in(0, a0, b0, isem)
        fire_in(1, a1, b1, osem)

        drain_in(a0, b0, isem)
        compute(a0, b0, o0)
        fire_out(gbase, o0, ws0)
        fire_in(2, a0, b0, isem)
        drain_in(a1, b1, osem)
        compute(a1, b1, o1)
        fire_out(gbase + 1, o1, ws1)
        fire_in(3, a1, b1, osem)

        def pair(p, _):
            ja = 2 * p
            drain_in(a0, b0, isem)
            drain_out(ws0)
            compute(a0, b0, o0)
            fire_out(gbase + ja, o0, ws0)
            fire_in(ja + 2, a0, b0, isem)
            drain_in(a1, b1, osem)
            drain_out(ws1)
            compute(a1, b1, o1)
            fire_out(gbase + ja + 1, o1, ws1)
            fire_in(ja + 3, a1, b1, osem)
            return _

        lax.fori_loop(1, _BLK_PER_W // 2, pair, None)

        drain_in(a0, b0, isem)
        drain_in(a1, b1, osem)
        drain_out(ws0)
        drain_out(ws1)

        @pl.when(wid < _EXTRA_BLOCKS)
        def _extra():
            g = _BLK_PER_W * _NW + wid
            pltpu.sync_copy(tt_hbm.at[pl.ds(0, 8), pl.ds(g * 128, 128)], a0)
            pltpu.sync_copy(tt_hbm.at[pl.ds(8, 8), pl.ds(g * 128, 128)], b0)
            compute_out(g, a0, b0, o0)

    return run


@functools.cache
def _build_gather():
    mesh = plsc.VectorSubcoreMesh(core_axis_name="c", subcore_axis_name="s")

    @functools.partial(
        pl.kernel,
        mesh=mesh,
        compiler_params=pltpu.CompilerParams(
            use_tc_tiling_on_sc=False, needs_layout_passes=False
        ),
        out_type=jax.ShapeDtypeStruct((_N_FEATURES, _EMBED * _BATCH), jnp.float32),
        scratch_types=[
            pltpu.VMEM((_N_FEATURES, _B_PER_W), jnp.int32),
            pltpu.VMEM((_B_PER_W, _EMBED), jnp.float32),
            pltpu.VMEM((_B_PER_W, _EMBED), jnp.float32),
            pltpu.VMEM((_EMBED * _B_PER_W,), jnp.float32),
            pltpu.VMEM((_EMBED * _B_PER_W,), jnp.float32),
            pltpu.SemaphoreType.DMA,
            pltpu.SemaphoreType.DMA,
            pltpu.SemaphoreType.DMA,
        ],
    )
    def run(
        xt_hbm, table_hbm, out_hbm, idx_all, rows_a, rows_b, t0, t1, sem, os0, os1
    ):
        wid = lax.axis_index("s") * 2 + lax.axis_index("c")
        b0 = wid * _B_PER_W

        pltpu.sync_copy(xt_hbm.at[:, pl.ds(b0, _B_PER_W)], idx_all)

        lane = lax.iota(jnp.int32, _LANES)
        lanepat = (
            lax.shift_right_logical(lane, 3) * (_B_PER_W * 8)
            + (lane & 7) * 128
        )

        def fire(f, dst):
            seg = table_hbm.at[pl.ds(f * _N_VALUES, _N_VALUES)]
            idx_row = idx_all.at[f]
            for g in range(_G_PER_F):
                pltpu.async_copy(
                    seg.at[idx_row.at[pl.ds(g * _G, _G)]],
                    dst.at[pl.ds(g * _G, _G)],
                    sem,
                )

        def drain(dst):
            for g in range(_G_PER_F):
                pltpu.make_async_copy(
                    table_hbm.at[pl.ds(0, _G)], dst.at[pl.ds(g * _G, _G)], sem
                ).wait()

        def transpose(src, dst):
            @plsc.parallel_loop(0, _B_PER_W, unroll=16)
            def _row(r):
                cb = (r // 128) * 1024 + r % 128
                v = src[r, :]
                plsc.store_scatter(dst, [lanepat + cb], v)

        def fire_out(f, src, osem):
            for et in range(_EMBED // 8):
                pltpu.async_copy(
                    src.at[pl.ds(et * (_B_PER_W * 8), _B_PER_W * 8)],
                    out_hbm.at[
                        f,
                        pl.ds(et * (_BATCH * 8) + wid * (_B_PER_W * 8), _B_PER_W * 8),
                    ],
                    osem,
                )

        def drain_out(osem):
            for et in range(_EMBED // 8):
                pltpu.make_async_copy(
                    t0.at[pl.ds(0, _B_PER_W * 8)],
                    out_hbm.at[0, pl.ds(0, _B_PER_W * 8)],
                    osem,
                ).wait()

        fire(0, rows_a)

        drain(rows_a)
        fire(1, rows_b)
        transpose(rows_a, t0)
        fire_out(0, t0, os0)
        drain(rows_b)
        fire(2, rows_a)
        transpose(rows_b, t1)
        fire_out(1, t1, os1)

        def pair(p, _):
            fa = 2 * p
            drain(rows_a)
            fire(fa + 1, rows_b)
            drain_out(os0)
            transpose(rows_a, t0)
            fire_out(fa, t0, os0)
            drain(rows_b)

            @pl.when(p < _N_FEATURES // 2 - 1)
            def _fire_next():
                fire(fa + 2, rows_a)

            drain_out(os1)
            transpose(rows_b, t1)
            fire_out(fa + 1, t1, os1)
            return _

        lax.fori_loop(1, _N_FEATURES // 2, pair, None)
        drain_out(os0)
        drain_out(os1)

    return run


def kernel(x, table):
    tt = jnp.transpose(table)
    tail = table[_FULL_BLOCKS * 128 :, :].reshape(-1)
    tbl_lin = _build_relayout()(tt, tail).reshape(_N_ROWS, _EMBED)

    xt = jnp.transpose(x)
    out_k = _build_gather()(xt, tbl_lin)
    k5 = out_k.reshape(_N_FEATURES, 2, _BATCH // 128, 8, 128)
    t = jnp.transpose(k5, (2, 4, 0, 1, 3))
    return t.reshape(_BATCH, _N_FEATURES, _EMBED)


# speedup vs baseline: 2.2008x; 1.0445x over previous
import functools

import jax
import jax.numpy as jnp
from jax import lax
from jax.experimental import pallas as pl
from jax.experimental.pallas import tpu as pltpu
from jax.experimental.pallas import tpu_sc as plsc

_N_FEATURES = 26
_N_VALUES = 38462
_EMBED = 16
_BATCH = 16384
_N_ROWS = _N_VALUES * _N_FEATURES

_LANES = 16
_G = 128
_NW = 32
_B_PER_W = _BATCH // _NW
_G_PER_F = _B_PER_W // _G

_FULL_BLOCKS = _N_ROWS // 128
_TAIL_ROWS = _N_ROWS - _FULL_BLOCKS * 128
_BLK_PER_W = 244
_EXTRA_BLOCKS = _FULL_BLOCKS - _BLK_PER_W * _NW


@functools.cache
def _build_relayout():
    mesh = plsc.VectorSubcoreMesh(core_axis_name="c", subcore_axis_name="s")

    @functools.partial(
        pl.kernel,
        mesh=mesh,
        compiler_params=pltpu.CompilerParams(
            use_tc_tiling_on_sc=True, needs_layout_passes=False
        ),
        out_type=jax.ShapeDtypeStruct((_N_ROWS * _EMBED,), jnp.float32),
        scratch_types=[
            pltpu.VMEM((8, 128), jnp.float32),
            pltpu.VMEM((8, 128), jnp.float32),
            pltpu.VMEM((8, 128), jnp.float32),
            pltpu.VMEM((8, 128), jnp.float32),
            pltpu.VMEM((2048,), jnp.float32),
            pltpu.VMEM((2048,), jnp.float32),
            pltpu.VMEM((_TAIL_ROWS * _EMBED,), jnp.float32),
            pltpu.SemaphoreType.DMA,
            pltpu.SemaphoreType.DMA,
            pltpu.SemaphoreType.DMA,
            pltpu.SemaphoreType.DMA,
        ],
    )
    def run(
        tt_hbm, tail_hbm, out_hbm, a0, b0, a1, b1, o0, o1, tbuf, isem, osem, ws0, ws1
    ):
        wid = lax.axis_index("s") * 2 + lax.axis_index("c")
        gbase = wid * _BLK_PER_W
        lane = lax.iota(jnp.int32, _LANES)
        rowpat = lane * _EMBED

        @pl.when(wid == 0)
        def _tail():
            pltpu.sync_copy(tail_hbm, tbuf)
            pltpu.sync_copy(
                tbuf, out_hbm.at[pl.ds(_FULL_BLOCKS * 2048, _TAIL_ROWS * _EMBED)]
            )

        def fire_in(j, bufa, bufb, sem):
            g = gbase + j
            pltpu.async_copy(tt_hbm.at[pl.ds(0, 8), pl.ds(g * 128, 128)], bufa, sem)
            pltpu.async_copy(tt_hbm.at[pl.ds(8, 8), pl.ds(g * 128, 128)], bufb, sem)

        def drain_in(bufa, bufb, sem):
            pltpu.make_async_copy(tt_hbm.at[pl.ds(0, 8), pl.ds(0, 128)], bufa, sem).wait()
            pltpu.make_async_copy(tt_hbm.at[pl.ds(0, 8), pl.ds(0, 128)], bufb, sem).wait()

        def compute(bufa, bufb, obuf):
            @plsc.parallel_loop(0, 64, unroll=16)
            def _iter(i):
                e = i // 8
                c0 = (i % 8) * _LANES
                idx = rowpat + (c0 * _EMBED + e)
                plsc.store_scatter(obuf, [idx], bufa[e, pl.ds(c0, _LANES)])
                plsc.store_scatter(obuf, [idx + 8], bufb[e, pl.ds(c0, _LANES)])

        def fire_out(g, obuf, sem):
            pltpu.async_copy(obuf, out_hbm.at[pl.ds(g * 2048, 2048)], sem)

        def drain_out(sem):
            pltpu.make_async_copy(o0, out_hbm.at[pl.ds(0, 2048)], sem).wait()

        def compute_out(g, bufa, bufb, obuf):
            compute(bufa, bufb, obuf)
            pltpu.sync_copy(obuf, out_hbm.at[pl.ds(g * 2048, 2048)])

        fire_in(0, a0, b0, isem)
        fire_in(1, a1, b1, osem)

        drain_in(a0, b0, isem)
        compute(a0, b0, o0)
        fire_out(gbase, o0, ws0)
        fire_in(2, a0, b0, isem)
        drain_in(a1, b1, osem)
        compute(a1, b1, o1)
        fire_out(gbase + 1, o1, ws1)
        fire_in(3, a1, b1, osem)

        def pair(p, _):
            ja = 2 * p
            drain_in(a0, b0, isem)
            drain_out(ws0)
            compute(a0, b0, o0)
            fire_out(gbase + ja, o0, ws0)
            fire_in(ja + 2, a0, b0, isem)
            drain_in(a1, b1, osem)
            drain_out(ws1)
            compute(a1, b1, o1)
            fire_out(gbase + ja + 1, o1, ws1)
            fire_in(ja + 3, a1, b1, osem)
            return _

        lax.fori_loop(1, _BLK_PER_W // 2, pair, None)

        drain_in(a0, b0, isem)
        drain_in(a1, b1, osem)
        drain_out(ws0)
        drain_out(ws1)

        @pl.when(wid < _EXTRA_BLOCKS)
        def _extra():
            g = _BLK_PER_W * _NW + wid
            pltpu.sync_copy(tt_hbm.at[pl.ds(0, 8), pl.ds(g * 128, 128)], a0)
            pltpu.sync_copy(tt_hbm.at[pl.ds(8, 8), pl.ds(g * 128, 128)], b0)
            compute_out(g, a0, b0, o0)

    return run


@functools.cache
def _build_gather():
    mesh = plsc.VectorSubcoreMesh(core_axis_name="c", subcore_axis_name="s")

    @functools.partial(
        pl.kernel,
        mesh=mesh,
        compiler_params=pltpu.CompilerParams(
            use_tc_tiling_on_sc=False, needs_layout_passes=False
        ),
        out_type=jax.ShapeDtypeStruct((_N_FEATURES, _EMBED * _BATCH), jnp.float32),
        scratch_types=[
            pltpu.VMEM((_N_FEATURES, _B_PER_W), jnp.int32),
            pltpu.VMEM((_B_PER_W, _EMBED), jnp.float32),
            pltpu.VMEM((_B_PER_W, _EMBED), jnp.float32),
            pltpu.VMEM((_EMBED * _B_PER_W,), jnp.float32),
            pltpu.VMEM((_EMBED * _B_PER_W,), jnp.float32),
            pltpu.SemaphoreType.DMA,
            pltpu.SemaphoreType.DMA,
            pltpu.SemaphoreType.DMA,
        ],
    )
    def run(
        xt_hbm, table_hbm, out_hbm, idx_all, rows_a, rows_b, t0, t1, sem, os0, os1
    ):
        wid = lax.axis_index("s") * 2 + lax.axis_index("c")
        b0 = wid * _B_PER_W

        pltpu.sync_copy(xt_hbm.at[:, pl.ds(b0, _B_PER_W)], idx_all)

        lane = lax.iota(jnp.int32, _LANES)
        lanepat = (
            lax.shift_right_logical(lane, 3) * (_B_PER_W * 8)
            + (lane & 7) * 128
        )

        def fire(f, dst):
            seg = table_hbm.at[pl.ds(f * _N_VALUES, _N_VALUES)]
            idx_row = idx_all.at[f]
            for g in range(_G_PER_F):
                pltpu.async_copy(
                    seg.at[idx_row.at[pl.ds(g * _G, _G)]],
                    dst.at[pl.ds(g * _G, _G)],
                    sem,
                )

        def drain(dst):
            for g in range(_G_PER_F):
                pltpu.make_async_copy(
                    table_hbm.at[pl.ds(0, _G)], dst.at[pl.ds(g * _G, _G)], sem
                ).wait()

        def transpose(src, dst):
            @plsc.parallel_loop(0, _B_PER_W, unroll=32)
            def _row(r):
                cb = (r // 128) * 1024 + r % 128
                v = src[r, :]
                plsc.store_scatter(dst, [lanepat + cb], v)

        def fire_out(f, src, osem):
            for et in range(_EMBED // 8):
                pltpu.async_copy(
                    src.at[pl.ds(et * (_B_PER_W * 8), _B_PER_W * 8)],
                    out_hbm.at[
                        f,
                        pl.ds(et * (_BATCH * 8) + wid * (_B_PER_W * 8), _B_PER_W * 8),
                    ],
                    osem,
                )

        def drain_out(osem):
            for et in range(_EMBED // 8):
                pltpu.make_async_copy(
                    t0.at[pl.ds(0, _B_PER_W * 8)],
                    out_hbm.at[0, pl.ds(0, _B_PER_W * 8)],
                    osem,
                ).wait()

        fire(0, rows_a)

        drain(rows_a)
        fire(1, rows_b)
        transpose(rows_a, t0)
        fire_out(0, t0, os0)
        drain(rows_b)
        fire(2, rows_a)
        transpose(rows_b, t1)
        fire_out(1, t1, os1)

        def pair(p, _):
            fa = 2 * p
            drain(rows_a)
            fire(fa + 1, rows_b)
            drain_out(os0)
            transpose(rows_a, t0)
            fire_out(fa, t0, os0)
            drain(rows_b)

            @pl.when(p < _N_FEATURES // 2 - 1)
            def _fire_next():
                fire(fa + 2, rows_a)

            drain_out(os1)
            transpose(rows_b, t1)
            fire_out(fa + 1, t1, os1)
            return _

        lax.fori_loop(1, _N_FEATURES // 2, pair, None)
        drain_out(os0)
        drain_out(os1)

    return run


def kernel(x, table):
    tt = jnp.transpose(table)
    tail = table[_FULL_BLOCKS * 128 :, :].reshape(-1)
    tbl_lin = _build_relayout()(tt, tail).reshape(_N_ROWS, _EMBED)

    xt = jnp.transpose(x)
    out_k = _build_gather()(xt, tbl_lin)
    k5 = out_k.reshape(_N_FEATURES, 2, _BATCH // 128, 8, 128)
    t = jnp.transpose(k5, (2, 4, 0, 1, 3))
    return t.reshape(_BATCH, _N_FEATURES, _EMBED)
